# baseline (speedup 1.0000x reference)
import functools

import jax
import jax.numpy as jnp
from jax import lax
from jax.experimental import pallas as pl
from jax.experimental.pallas import tpu as pltpu
from jax.experimental.pallas import tpu_sc as plsc

T = 8
NCLIP = 8
C = 512
HW = 784
ROWS = NCLIP * T * C
G = 4
GC = C // G
CH = 16


def _pool_body(x_ref, o_ref):
    o_ref[...] = jnp.sum(x_ref[...], axis=-1, keepdims=True) * (1.0 / HW)


def _pool(x3):
    out = pl.pallas_call(
        _pool_body,
        grid=(64,),
        in_specs=[pl.BlockSpec((1, C, HW), lambda i: (i, 0, 0))],
        out_specs=pl.BlockSpec((1, C, 1), lambda i: (i, 0, 0)),
        out_shape=jax.ShapeDtypeStruct((64, C, 1), jnp.float32),
    )(x3)
    return out.reshape(64, C)


def _coef_body(pooled_ref, wall_ref, fbig_ref, fcb_ref, lbig_ref, lastb_ref,
               misc_ref, idx0_ref, idx1_ref, coef0_ref, coef1_ref):
    P = pooled_ref[...]
    M = jnp.dot(P, wall_ref[...], preferred_element_type=jnp.float32)

    ri = lax.broadcasted_iota(jnp.int32, (64, 64), 0)
    rj = lax.broadcasted_iota(jnp.int32, (64, 64), 1)
    sm = ((rj == ri - 1) & (ri % 8 != 0)).astype(jnp.float32)
    sp = ((rj == ri + 1) & (ri % 8 != 7)).astype(jnp.float32)
    Md = jnp.dot(sm, M, preferred_element_type=jnp.float32)
    Mu = jnp.dot(sp, M, preferred_element_type=jnp.float32)

    conv_b = misc_ref[0:1, 0:1]
    wconv_b0 = misc_ref[0:1, 1:2]
    wconv_b1 = misc_ref[0:1, 2:3]

    xb = Md[:, 0:1] + M[:, 1:2] + Mu[:, 2:3] + conv_b
    xw0 = Md[:, 3:4] + M[:, 4:5] + Mu[:, 5:6] + wconv_b0
    xw1 = Md[:, 6:7] + M[:, 7:8] + Mu[:, 8:9] + wconv_b1
    xweight0 = 2.0 * jax.nn.sigmoid(xw0)
    xweight1 = 2.0 * jax.nn.sigmoid(xw1)

    y = jnp.dot(fbig_ref[...], xb, preferred_element_type=jnp.float32)
    y = jax.nn.relu(y + fcb_ref[...])
    z = jnp.dot(lbig_ref[...], y, preferred_element_type=jnp.float32)
    z = z + lastb_ref[...]
    z = 4.0 * (jax.nn.sigmoid(z) - 0.5)

    ei = lax.broadcasted_iota(jnp.int32, (64, 16), 0)
    ek = lax.broadcasted_iota(jnp.int32, (64, 16), 1)
    e_even = (ek == 2 * (ei // 8)).astype(jnp.float32)
    e_odd = (ek == 2 * (ei // 8) + 1).astype(jnp.float32)
    u = jnp.dot(e_even, z, preferred_element_type=jnp.float32)
    v = jnp.dot(e_odd, z, preferred_element_type=jnp.float32)

    cg = lax.broadcasted_iota(jnp.int32, (1, C), 1) // GC
    m0 = (cg == 0).astype(jnp.float32)
    m1 = (cg == 1).astype(jnp.float32)
    m2 = (cg == 2).astype(jnp.float32)
    m3 = (cg == 3).astype(jnp.float32)

    B = u * (m0 - m2) + v * (m1 - m3)
    Bf = jnp.floor(B)
    b0 = Bf.astype(jnp.int32)
    w0 = 1.0 - (B - Bf)
    w1 = B - Bf

    xw4 = xweight0 * (m0 + m2) + xweight1 * (m1 + m3)

    tmat = lax.broadcasted_iota(jnp.int32, (64, C), 0) % 8
    nbase = lax.broadcasted_iota(jnp.int32, (64, C), 0) - tmat
    cidx = lax.broadcasted_iota(jnp.int32, (64, C), 1)

    t0 = tmat + b0
    valid0 = ((t0 >= 0) & (t0 < T)).astype(jnp.float32)
    t0c = jnp.clip(t0, 0, T - 1)
    t1 = t0 + 1
    valid1 = ((t1 >= 0) & (t1 < T)).astype(jnp.float32)
    t1c = jnp.clip(t1, 0, T - 1)

    idx0_ref[...] = (nbase + t0c) * C + cidx
    idx1_ref[...] = (nbase + t1c) * C + cidx
    coef0_ref[...] = xw4 * w0 * valid0
    coef1_ref[...] = xw4 * w1 * valid1


def _coefs(pooled, wall, fbig, fcb, lbig, lastb, misc):
    return pl.pallas_call(
        _coef_body,
        out_shape=(
            jax.ShapeDtypeStruct((64, C), jnp.int32),
            jax.ShapeDtypeStruct((64, C), jnp.int32),
            jax.ShapeDtypeStruct((64, C), jnp.float32),
            jax.ShapeDtypeStruct((64, C), jnp.float32),
        ),
    )(pooled, wall, fbig, fcb, lbig, lastb, misc)


def _sc_body(nc, rpw, x_hbm, idx0_hbm, idx1_hbm, coef0_hbm, coef1_hbm, out_hbm,
             idx0_v, idx1_v, c0_v, c1_v, buf0, buf1, obuf, sem0, sem1):
    wid = lax.axis_index("s") * nc + lax.axis_index("c")
    base = wid * rpw

    pltpu.sync_copy(idx0_hbm.at[pl.ds(base, rpw)], idx0_v)
    pltpu.sync_copy(idx1_hbm.at[pl.ds(base, rpw)], idx1_v)
    pltpu.sync_copy(coef0_hbm.at[pl.ds(base, rpw)], c0_v)
    pltpu.sync_copy(coef1_hbm.at[pl.ds(base, rpw)], c1_v)

    def chunk_body(i, carry):
        off = i * CH
        iv0 = idx0_v[pl.ds(off, CH)]
        iv1 = idx1_v[pl.ds(off, CH)]
        cp0 = pltpu.async_copy(x_hbm.at[iv0], buf0, sem0)
        cp1 = pltpu.async_copy(x_hbm.at[iv1], buf1, sem1)
        cp0.wait()
        cp1.wait()

        def row_body(r, rc):
            c0 = c0_v[off + r]
            c1 = c1_v[off + r]
            for j in range(HW // 16):
                sl = pl.ds(j * 16, 16)
                obuf[r, sl] = c0 * buf0[r, sl] + c1 * buf1[r, sl]
            return rc

        lax.fori_loop(0, CH, row_body, 0)
        pltpu.sync_copy(obuf, out_hbm.at[pl.ds(base + off, CH)])
        return carry

    lax.fori_loop(0, rpw // CH, chunk_body, 0)


def _gather_lerp(x2d, idx0, idx1, coef0b, coef1b):
    info = plsc.get_sparse_core_info()
    nw = info.num_cores * info.num_subcores
    rpw = ROWS // nw
    mesh = plsc.VectorSubcoreMesh(core_axis_name="c", subcore_axis_name="s")
    fn = pl.kernel(
        functools.partial(_sc_body, info.num_cores, rpw),
        out_type=jax.ShapeDtypeStruct((ROWS, HW), jnp.float32),
        mesh=mesh,
        scratch_types=[
            pltpu.VMEM((rpw,), jnp.int32),
            pltpu.VMEM((rpw,), jnp.int32),
            pltpu.VMEM((rpw, 16), jnp.float32),
            pltpu.VMEM((rpw, 16), jnp.float32),
            pltpu.VMEM((CH, HW), jnp.float32),
            pltpu.VMEM((CH, HW), jnp.float32),
            pltpu.VMEM((CH, HW), jnp.float32),
            pltpu.SemaphoreType.DMA,
            pltpu.SemaphoreType.DMA,
        ],
        compiler_params=pltpu.CompilerParams(use_tc_tiling_on_sc=False),
    )
    return fn(x2d, idx0, idx1, coef0b, coef1b)


def kernel(x, conv_w, conv_b, fc_w, fc_b, last_w, last_b, wconv_w, wconv_b):
    nt, c, h, w = x.shape
    x3 = x.reshape(nt, c, h * w)

    pooled = _pool(x3)

    wall = jnp.zeros((C, 16), jnp.float32)
    wall = wall.at[:, 0:3].set(conv_w[0].astype(jnp.float32))
    wall = wall.at[:, 3:6].set(wconv_w[0].astype(jnp.float32))
    wall = wall.at[:, 6:9].set(wconv_w[1].astype(jnp.float32))
    fbig = jnp.kron(jnp.eye(8, dtype=jnp.float32), fc_w)
    lbig = jnp.kron(jnp.eye(8, dtype=jnp.float32), last_w)
    fcb = jnp.tile(fc_b, 8).reshape(64, 1)
    lastb = jnp.tile(last_b, 8).reshape(16, 1)
    misc = jnp.zeros((1, 128), jnp.float32)
    misc = misc.at[0, 0].set(conv_b[0])
    misc = misc.at[0, 1].set(wconv_b[0])
    misc = misc.at[0, 2].set(wconv_b[1])

    idx0, idx1, coef0, coef1 = _coefs(pooled, wall, fbig, fcb, lbig, lastb, misc)

    idx0 = idx0.reshape(ROWS)
    idx1 = idx1.reshape(ROWS)
    coef0b = jnp.broadcast_to(coef0.reshape(ROWS, 1), (ROWS, 16))
    coef1b = jnp.broadcast_to(coef1.reshape(ROWS, 1), (ROWS, 16))

    x2d = x3.reshape(ROWS, HW)
    out2d = _gather_lerp(x2d, idx0, idx1, coef0b, coef1b)
    return out2d.reshape(nt, c, h, w)
